# baseline (speedup 1.0000x reference)
import functools
import numpy as np
from math import factorial

import jax
import jax.numpy as jnp
from jax import lax
from jax.experimental import pallas as pl
from jax.experimental.pallas import tpu as pltpu
from jax.experimental.pallas import tpu_sc as plsc

_LMAX = 5
_NTAU = 16
_B = 32
_NROWS = 457
_ROW_F32 = 512
_CHUNK_ROWS = 64
_BUFSZ = _CHUNK_ROWS * _ROW_F32


def _cg_coef(l1, l2, l, m1, m2):
    m = m1 + m2
    if abs(m) > l:
        return 0.0
    pref = (2 * l + 1) * factorial(l + l1 - l2) * factorial(l - l1 + l2) * factorial(l1 + l2 - l) / factorial(l1 + l2 + l + 1)
    pref *= factorial(l + m) * factorial(l - m) * factorial(l1 - m1) * factorial(l1 + m1) * factorial(l2 - m2) * factorial(l2 + m2)
    kmin = max(0, l2 - l - m1, l1 + m2 - l)
    kmax = min(l1 + l2 - l, l1 - m1, l2 + m2)
    s = 0.0
    for k in range(kmin, kmax + 1):
        s += (-1) ** k / (factorial(k) * factorial(l1 + l2 - l - k) * factorial(l1 - m1 - k) * factorial(l2 + m2 - k) * factorial(l - l2 + m1 + k) * factorial(l - l1 - m2 + k))
    return float(np.sqrt(pref) * s)


def _build_terms():
    lt = []
    for l in range(_LMAX + 1):
        pairs = []
        for l1 in range(_LMAX + 1):
            for l2 in range(l1, _LMAX + 1):
                if l2 - l1 <= l <= l1 + l2:
                    pairs.append((l1, l2))
        lt.append(sorted(pairs))
    cum_el = 16 * np.concatenate([[0], (1 + 2 * np.arange(_LMAX + 1)).cumsum()]).astype(int)
    rows = []
    for l in range(_LMAX + 1):
        mats = {}
        for (l1, l2) in lt[l]:
            M = np.zeros((2 * l + 1, 2 * l1 + 1, 2 * l2 + 1), dtype=np.float64)
            for m1 in range(-l1, l1 + 1):
                for m2 in range(-l2, l2 + 1):
                    if abs(m1 + m2) <= l:
                        M[m1 + m2 + l, m1 + l1, m2 + l2] = _cg_coef(l1, l2, l, m1, m2)
            mats[(l1, l2)] = M
        for a in range(2 * l + 1):
            for (l1, l2) in lt[l]:
                M = mats[(l1, l2)]
                terms = []
                for x in range(2 * l1 + 1):
                    y = (a - l) - (x - l1) + l2
                    if 0 <= y <= 2 * l2 and M[a, x, y] != 0.0:
                        terms.append((int(cum_el[l1] + 16 * x), int(cum_el[l2] + 16 * y), float(M[a, x, y])))
                rows.append(terms)
    assert len(rows) == _NROWS
    assert all(len(r) >= 1 for r in rows)
    return rows


def _build_tables():
    rows = _build_terms()
    n_chunks = (_NROWS + _CHUNK_ROWS - 1) // _CHUNK_ROWS
    f_rec, r_rec = [], []
    f_starts, r_starts = [0], [0]

    def rec(i1, i2, ob, c):
        cb = int(np.float32(c).view(np.int32))
        return [i1, i2, ob, cb, 0, 0, 0, 0]

    for c in range(n_chunks):
        r0 = c * _CHUNK_ROWS
        r1 = min(_NROWS, r0 + _CHUNK_ROWS)
        par_off = (c % 2) * _BUFSZ
        for k in range(r0, r1):
            ob = par_off + (k - r0) * _ROW_F32
            first, *rest = rows[k]
            f_rec.append(rec(first[0], first[1], ob, first[2]))
            for t in rest:
                r_rec.append(rec(t[0], t[1], ob, t[2]))
        f_starts.append(len(f_rec))
        r_starts.append(len(r_rec))
    f_rec.append(rec(0, 0, 2 * _BUFSZ, 0.0))
    r_rec.append(rec(0, 0, 2 * _BUFSZ, 0.0))
    f_arr = np.asarray(f_rec, dtype=np.int32).reshape(-1)
    r_arr = np.asarray(r_rec, dtype=np.int32).reshape(-1)
    return n_chunks, f_starts, r_starts, f_arr, r_arr


_NCHUNKS, _FSTARTS, _RSTARTS, _FREC, _RREC = _build_tables()


def _sc_body(act_hbm, frec_h, rrec_h,
             out_hbm,
             zre_v, zim_v, frec_v, rrec_v,
             buf_v, sem0, sem1):
    wid = lax.axis_index("s") * 2 + lax.axis_index("c")
    pltpu.sync_copy(act_hbm.at[wid, 0], zre_v)
    pltpu.sync_copy(act_hbm.at[wid, 1], zim_v)
    pltpu.sync_copy(frec_h, frec_v)
    pltpu.sync_copy(rrec_h, rrec_v)

    iota2 = lax.iota(jnp.int32, 16) * 2

    def make_body(rec_v, add):
        def body(j, carry):
            reci = rec_v[pl.ds(8 * j, 16)]
            i1 = reci[0]
            i2 = reci[1]
            ob = reci[2]
            cc = lax.bitcast_convert_type(reci[3], jnp.float32)
            vf2r = zre_v[pl.ds(i2, 16)]
            vf2i = zim_v[pl.ds(i2, 16)]
            gr = cc * zre_v[pl.ds(i1, 16)]
            gi = cc * zim_v[pl.ds(i1, 16)]
            for s in range(16):
                ar = gr[s]
                ai = gi[s]
                cr = ar * vf2r - ai * vf2i
                ci = ar * vf2i + ai * vf2r
                idx = iota2 + (ob + 32 * s)
                if add:
                    plsc.addupdate_scatter(buf_v, [idx], cr)
                    plsc.addupdate_scatter(buf_v, [idx + 1], ci)
                else:
                    plsc.store_scatter(buf_v, [idx], cr)
                    plsc.store_scatter(buf_v, [idx + 1], ci)
            return carry
        return body

    first_body = make_body(frec_v, add=False)
    rest_body = make_body(rrec_v, add=True)

    sems = (sem0, sem1)
    copies = [None, None]
    for c in range(_NCHUNKS):
        par = c % 2
        if copies[par] is not None:
            copies[par].wait()
            copies[par] = None
        lax.fori_loop(_FSTARTS[c], _FSTARTS[c + 1], first_body, 0)
        lax.fori_loop(_RSTARTS[c], _RSTARTS[c + 1], rest_body, 0)
        r0 = c * _CHUNK_ROWS
        nrow = min(_NROWS, r0 + _CHUNK_ROWS) - r0
        cp = pltpu.async_copy(
            buf_v.at[pl.ds(par * _BUFSZ, nrow * _ROW_F32)],
            out_hbm.at[wid, pl.ds(r0 * _ROW_F32, nrow * _ROW_F32)],
            sems[par])
        copies[par] = cp
    for cp in copies:
        if cp is not None:
            cp.wait()


def kernel(activations):
    B = activations.shape[0]
    act = activations.transpose(0, 2, 1)
    sc_call = functools.partial(
        pl.kernel,
        out_type=jax.ShapeDtypeStruct((B, _NROWS * _ROW_F32), jnp.float32),
        mesh=plsc.VectorSubcoreMesh(core_axis_name="c", subcore_axis_name="s"),
        compiler_params=pltpu.CompilerParams(needs_layout_passes=False),
        scratch_types=[
            pltpu.VMEM((576,), jnp.float32),
            pltpu.VMEM((576,), jnp.float32),
            pltpu.VMEM((_FREC.size,), jnp.int32),
            pltpu.VMEM((_RREC.size,), jnp.int32),
            pltpu.VMEM((2 * _BUFSZ,), jnp.float32),
            pltpu.SemaphoreType.DMA,
            pltpu.SemaphoreType.DMA,
        ],
    )(_sc_body)
    out = sc_call(act, _FREC, _RREC)
    return out.reshape(B, _NROWS * 256, 2)


# speedup vs baseline: 2.1438x; 2.1438x over previous
import functools
import numpy as np
from math import factorial

import jax
import jax.numpy as jnp
from jax import lax
from jax.experimental import pallas as pl
from jax.experimental.pallas import tpu as pltpu
from jax.experimental.pallas import tpu_sc as plsc

_LMAX = 5
_NTAU = 16
_B = 32
_NROWS = 457
_ROW_F32 = 512
_CHUNK_ROWS = 64
_BUFSZ = _CHUNK_ROWS * _ROW_F32


def _cg_coef(l1, l2, l, m1, m2):
    m = m1 + m2
    if abs(m) > l:
        return 0.0
    pref = (2 * l + 1) * factorial(l + l1 - l2) * factorial(l - l1 + l2) * factorial(l1 + l2 - l) / factorial(l1 + l2 + l + 1)
    pref *= factorial(l + m) * factorial(l - m) * factorial(l1 - m1) * factorial(l1 + m1) * factorial(l2 - m2) * factorial(l2 + m2)
    kmin = max(0, l2 - l - m1, l1 + m2 - l)
    kmax = min(l1 + l2 - l, l1 - m1, l2 + m2)
    s = 0.0
    for k in range(kmin, kmax + 1):
        s += (-1) ** k / (factorial(k) * factorial(l1 + l2 - l - k) * factorial(l1 - m1 - k) * factorial(l2 + m2 - k) * factorial(l - l2 + m1 + k) * factorial(l - l1 - m2 + k))
    return float(np.sqrt(pref) * s)


def _build_terms():
    lt = []
    for l in range(_LMAX + 1):
        pairs = []
        for l1 in range(_LMAX + 1):
            for l2 in range(l1, _LMAX + 1):
                if l2 - l1 <= l <= l1 + l2:
                    pairs.append((l1, l2))
        lt.append(sorted(pairs))
    cum_el = 16 * np.concatenate([[0], (1 + 2 * np.arange(_LMAX + 1)).cumsum()]).astype(int)
    rows = []
    for l in range(_LMAX + 1):
        mats = {}
        for (l1, l2) in lt[l]:
            M = np.zeros((2 * l + 1, 2 * l1 + 1, 2 * l2 + 1), dtype=np.float64)
            for m1 in range(-l1, l1 + 1):
                for m2 in range(-l2, l2 + 1):
                    if abs(m1 + m2) <= l:
                        M[m1 + m2 + l, m1 + l1, m2 + l2] = _cg_coef(l1, l2, l, m1, m2)
            mats[(l1, l2)] = M
        for a in range(2 * l + 1):
            for (l1, l2) in lt[l]:
                M = mats[(l1, l2)]
                terms = []
                for x in range(2 * l1 + 1):
                    y = (a - l) - (x - l1) + l2
                    if 0 <= y <= 2 * l2 and M[a, x, y] != 0.0:
                        terms.append((int(cum_el[l1] + 16 * x), int(cum_el[l2] + 16 * y), float(M[a, x, y])))
                rows.append(terms)
    assert len(rows) == _NROWS
    assert all(len(r) >= 1 for r in rows)
    return rows


def _build_tables():
    rows = _build_terms()
    n_chunks = (_NROWS + _CHUNK_ROWS - 1) // _CHUNK_ROWS
    f_rec, r_rec = [], []
    f_starts, r_starts = [0], [0]

    def rec(i1, i2, ob, c):
        cb = int(np.float32(c).view(np.int32))
        return [i1, i2, ob, cb, 0, 0, 0, 0]

    for c in range(n_chunks):
        r0 = c * _CHUNK_ROWS
        r1 = min(_NROWS, r0 + _CHUNK_ROWS)
        par_off = (c % 2) * _CHUNK_ROWS * 2
        for k in range(r0, r1):
            ob = par_off + (k - r0) * 2
            first, *rest = rows[k]
            f_rec.append(rec(first[0], first[1], ob, first[2]))
            for t in rest:
                r_rec.append(rec(t[0], t[1], ob, t[2]))
        f_starts.append(len(f_rec))
        r_starts.append(len(r_rec))
    f_rec.append(rec(0, 0, 0, 0.0))
    r_rec.append(rec(0, 0, 0, 0.0))
    f_arr = np.asarray(f_rec, dtype=np.int32).reshape(-1)
    r_arr = np.asarray(r_rec, dtype=np.int32).reshape(-1)
    return n_chunks, f_starts, r_starts, f_arr, r_arr


_NCHUNKS, _FSTARTS, _RSTARTS, _FREC, _RREC = _build_tables()


def _sc_body(act_hbm, frec_h, rrec_h,
             out_hbm,
             zre_v, zim_v, frec_v, rrec_v,
             buf_v, sem0, sem1):
    wid = lax.axis_index("s") * 2 + lax.axis_index("c")
    pltpu.sync_copy(act_hbm.at[wid, 0], zre_v)
    pltpu.sync_copy(act_hbm.at[wid, 1], zim_v)
    pltpu.sync_copy(frec_h, frec_v)
    pltpu.sync_copy(rrec_h, rrec_v)

    def make_body(rec_v, add):
        def body(j, carry):
            reci = rec_v[pl.ds(8 * j, 16)]
            i1 = reci[0]
            i2 = reci[1]
            ob = reci[2]
            cc = lax.bitcast_convert_type(reci[3], jnp.float32)
            vf2r = zre_v[pl.ds(i2, 16)]
            vf2i = zim_v[pl.ds(i2, 16)]
            gr = cc * zre_v[pl.ds(i1, 16)]
            gi = cc * zim_v[pl.ds(i1, 16)]
            for s in range(16):
                ar = gr[s]
                ai = gi[s]
                cr = ar * vf2r - ai * vf2i
                ci = ar * vf2i + ai * vf2r
                row = ob + s // 8
                t0 = 16 * (s % 8)
                if add:
                    plsc.addupdate(buf_v.at[row, 0, pl.ds(t0, 16)], cr)
                    plsc.addupdate(buf_v.at[row, 1, pl.ds(t0, 16)], ci)
                else:
                    buf_v[row, 0, pl.ds(t0, 16)] = cr
                    buf_v[row, 1, pl.ds(t0, 16)] = ci
            return carry
        return body

    first_body = make_body(frec_v, add=False)
    rest_body = make_body(rrec_v, add=True)

    sems = (sem0, sem1)
    copies = [None, None]
    for c in range(_NCHUNKS):
        par = c % 2
        if copies[par] is not None:
            copies[par].wait()
            copies[par] = None
        lax.fori_loop(_FSTARTS[c], _FSTARTS[c + 1], first_body, 0)
        lax.fori_loop(_RSTARTS[c], _RSTARTS[c + 1], rest_body, 0)
        r0 = c * _CHUNK_ROWS
        nrow = min(_NROWS, r0 + _CHUNK_ROWS) - r0
        cp = pltpu.async_copy(
            buf_v.at[pl.ds(par * _CHUNK_ROWS * 2, nrow * 2), :, :],
            out_hbm.at[wid, pl.ds(r0 * 2, nrow * 2), :, :],
            sems[par])
        copies[par] = cp
    for cp in copies:
        if cp is not None:
            cp.wait()


def kernel(activations):
    B = activations.shape[0]
    act = activations.transpose(0, 2, 1)
    sc_call = functools.partial(
        pl.kernel,
        out_type=jax.ShapeDtypeStruct((B, _NROWS * 2, 2, 128), jnp.float32),
        mesh=plsc.VectorSubcoreMesh(core_axis_name="c", subcore_axis_name="s"),
        compiler_params=pltpu.CompilerParams(needs_layout_passes=False, use_tc_tiling_on_sc=False),
        scratch_types=[
            pltpu.VMEM((576,), jnp.float32),
            pltpu.VMEM((576,), jnp.float32),
            pltpu.VMEM((_FREC.size,), jnp.int32),
            pltpu.VMEM((_RREC.size,), jnp.int32),
            pltpu.VMEM((2 * _CHUNK_ROWS * 2, 2, 128), jnp.float32),
            pltpu.SemaphoreType.DMA,
            pltpu.SemaphoreType.DMA,
        ],
    )(_sc_body)
    out4 = sc_call(act, _FREC, _RREC)
    return out4.transpose(0, 1, 3, 2).reshape(B, _NROWS * 256, 2)
